# Initial kernel scaffold; baseline (speedup 1.0000x reference)
import functools

import jax
import jax.numpy as jnp
from jax import lax
from jax.experimental import pallas as pl
from jax.experimental.pallas import tpu as pltpu
from jax.experimental.pallas import tpu_sc as plsc

N_NODES = 50000
DIM = 256
N_GRAPHS = 500

ROW_BLK = 512
CH = 128
N_PAD = 50176
N_CHUNKS = N_PAD // CH
N_WORKERS = 32
GPAD = 512


def _gate_body(x_ref, w1_ref, b1_ref, w2_ref, b2_ref, o_ref):
    x = x_ref[...]
    h1 = jnp.dot(x, w1_ref[...], preferred_element_type=jnp.float32) + b1_ref[...]
    h2 = jnp.dot(x, w2_ref[...], preferred_element_type=jnp.float32) + b2_ref[...]
    o_ref[...] = h1 * jax.nn.sigmoid(h2)


def _gated_h(x, W1, b1, W2, b2):
    grid = (N_PAD // ROW_BLK,)
    return pl.pallas_call(
        _gate_body,
        grid=grid,
        in_specs=[
            pl.BlockSpec((ROW_BLK, DIM), lambda i: (i, 0)),
            pl.BlockSpec((DIM, DIM), lambda i: (0, 0)),
            pl.BlockSpec((1, DIM), lambda i: (0, 0)),
            pl.BlockSpec((DIM, DIM), lambda i: (0, 0)),
            pl.BlockSpec((1, DIM), lambda i: (0, 0)),
        ],
        out_specs=pl.BlockSpec((ROW_BLK, DIM), lambda i: (i, 0)),
        out_shape=jax.ShapeDtypeStruct((N_PAD, DIM), jnp.float32),
    )(x, W1, b1.reshape(1, DIM), W2, b2.reshape(1, DIM))


def _seg_body(h_hbm, ids_hbm, ones_hbm, zsum_hbm, zcnt_hbm,
              z1_out, cnt_out,
              hrows_v, idx_v, ones_v, zb_v, cb_v, z1_sh, cnt_sh):
    cid = lax.axis_index("c")
    sid = lax.axis_index("s")
    wid = sid * 2 + cid

    @pl.when(sid == 0)
    def _():
        pltpu.sync_copy(zsum_hbm, z1_sh)
        pltpu.sync_copy(zcnt_hbm, cnt_sh)

    pltpu.sync_copy(ones_hbm, ones_v)
    plsc.subcore_barrier()

    def chunk(c):
        pltpu.sync_copy(ids_hbm.at[c], idx_v)
        pltpu.sync_copy(h_hbm.at[pl.ds(c * CH, CH)], hrows_v)
        pltpu.sync_copy(hrows_v, z1_sh.at[idx_v], add=True)
        pltpu.sync_copy(ones_v, cnt_sh.at[idx_v], add=True)

    n_loops = (N_CHUNKS + N_WORKERS - 1) // N_WORKERS
    for j in range(n_loops):
        c = wid + N_WORKERS * j
        if (j + 1) * N_WORKERS <= N_CHUNKS:
            chunk(c)
        else:
            @pl.when(c < N_CHUNKS)
            def _():
                chunk(c)

    plsc.subcore_barrier()

    rows = GPAD // 16
    pltpu.sync_copy(z1_sh.at[pl.ds(sid * rows, rows)], zb_v)
    pltpu.sync_copy(cnt_sh.at[pl.ds(sid * rows, rows)], cb_v)
    pltpu.sync_copy(zb_v, z1_out.at[cid, pl.ds(sid * rows, rows)])
    pltpu.sync_copy(cb_v, cnt_out.at[cid, pl.ds(sid * rows, rows)])


def _segment_sums(h_pad, ids2d):
    ones = jnp.ones((CH, 16), jnp.float32)
    zsum = jnp.zeros((GPAD, DIM), jnp.float32)
    zcnt = jnp.zeros((GPAD, 16), jnp.float32)
    mesh = plsc.VectorSubcoreMesh(core_axis_name="c", subcore_axis_name="s")
    f = pl.kernel(
        _seg_body,
        out_type=(
            jax.ShapeDtypeStruct((2, GPAD, DIM), jnp.float32),
            jax.ShapeDtypeStruct((2, GPAD, 16), jnp.float32),
        ),
        mesh=mesh,
        scratch_types=[
            pltpu.VMEM((CH, DIM), jnp.float32),
            pltpu.VMEM((CH,), jnp.int32),
            pltpu.VMEM((CH, 16), jnp.float32),
            pltpu.VMEM((GPAD // 16, DIM), jnp.float32),
            pltpu.VMEM((GPAD // 16, 16), jnp.float32),
            pltpu.VMEM_SHARED((GPAD, DIM), jnp.float32),
            pltpu.VMEM_SHARED((GPAD, 16), jnp.float32),
        ],
    )
    return f(h_pad, ids2d, ones, zsum, zcnt)


def _final_body(z1p_ref, cnt_ref, w3a_ref, w3b_ref, b3_ref, o_ref):
    z1 = z1p_ref[0] + z1p_ref[1]
    cnt = cnt_ref[0][:, 0:1] + cnt_ref[1][:, 0:1]
    z2 = z1 / jnp.maximum(cnt, 1.0)
    o_ref[...] = (
        jnp.dot(z1, w3a_ref[...], preferred_element_type=jnp.float32)
        + jnp.dot(z2, w3b_ref[...], preferred_element_type=jnp.float32)
        + b3_ref[...]
    )


def _readout(z1p, cntp, W3, b3):
    out = pl.pallas_call(
        _final_body,
        out_shape=jax.ShapeDtypeStruct((GPAD, DIM), jnp.float32),
    )(z1p, cntp, W3[:DIM], W3[DIM:], b3.reshape(1, DIM))
    return out[:N_GRAPHS]


def kernel(x, node2graph, W1, b1, W2, b2, W3, b3):
    h_pad = _gated_h(x, W1, b1, W2, b2)
    ids = node2graph.astype(jnp.int32)
    ids_pad = jnp.concatenate(
        [ids, jnp.full((N_PAD - N_NODES,), N_GRAPHS, jnp.int32)]
    ).reshape(N_CHUNKS, CH)
    z1p, cntp = _segment_sums(h_pad, ids_pad)
    return _readout(z1p, cntp, W3, b3)


# speedup vs baseline: 2.0898x; 2.0898x over previous
import jax
import jax.numpy as jnp
from jax import lax
from jax.experimental import pallas as pl
from jax.experimental.pallas import tpu as pltpu
from jax.experimental.pallas import tpu_sc as plsc

N_NODES = 50000
DIM = 256
HALF = 128
N_GRAPHS = 500

ROW_BLK = 512
CH = 128
N_PAD = 50176
N_CHUNKS = N_PAD // CH
N_WORKERS = 32
GPAD = 504
CNT_ROWS = 64


def _gate_body(x_ref, w1_ref, b1_ref, w2_ref, b2_ref, lo_ref, hi_ref):
    x = x_ref[...]
    h1 = jnp.dot(x, w1_ref[...], preferred_element_type=jnp.float32) + b1_ref[...]
    h2 = jnp.dot(x, w2_ref[...], preferred_element_type=jnp.float32) + b2_ref[...]
    h = h1 * jax.nn.sigmoid(h2)
    lo_ref[...] = h[:, :HALF]
    hi_ref[...] = h[:, HALF:]


def _gated_h(x, W1, b1, W2, b2):
    grid = (N_PAD // ROW_BLK,)
    return pl.pallas_call(
        _gate_body,
        grid=grid,
        in_specs=[
            pl.BlockSpec((ROW_BLK, DIM), lambda i: (i, 0)),
            pl.BlockSpec((DIM, DIM), lambda i: (0, 0)),
            pl.BlockSpec((1, DIM), lambda i: (0, 0)),
            pl.BlockSpec((DIM, DIM), lambda i: (0, 0)),
            pl.BlockSpec((1, DIM), lambda i: (0, 0)),
        ],
        out_specs=[
            pl.BlockSpec((ROW_BLK, HALF), lambda i: (i, 0)),
            pl.BlockSpec((ROW_BLK, HALF), lambda i: (i, 0)),
        ],
        out_shape=[
            jax.ShapeDtypeStruct((N_PAD, HALF), jnp.float32),
            jax.ShapeDtypeStruct((N_PAD, HALF), jnp.float32),
        ],
    )(x, W1, b1.reshape(1, DIM), W2, b2.reshape(1, DIM))


def _seg_body(hlo_hbm, hhi_hbm, ids_hbm,
              part_out, cnt_out,
              acc_v, hst_v, ids_v, cnt_v):
    cid = lax.axis_index("c")
    sid = lax.axis_index("s")
    wid = sid * 2 + cid
    nj = jnp.where(wid < N_CHUNKS - 12 * N_WORKERS, 13, 12)
    ones16 = jnp.ones((16,), jnp.float32)
    zero16 = jnp.zeros((16,), jnp.float32)

    def zero_acc(_i, carry):
        for kk in range(HALF // 16):
            acc_v[_i, pl.ds(kk * 16, 16)] = zero16
        return carry

    def half_pass(src_hbm, do_cnt):
        lax.fori_loop(0, GPAD, zero_acc, 0)

        def do_chunk(j, carry):
            c = wid + N_WORKERS * j
            pltpu.sync_copy(ids_hbm.at[c], ids_v)
            pltpu.sync_copy(src_hbm.at[pl.ds(c * CH, CH)], hst_v)

            def group_body(rg, carry2):
                ids16 = ids_v[pl.ds(rg * 16, 16)]
                for l in range(16):
                    g = ids16[l]
                    r = rg * 16 + l
                    for kk in range(HALF // 16):
                        v = hst_v[r, pl.ds(kk * 16, 16)]
                        plsc.addupdate(acc_v.at[g, pl.ds(kk * 16, 16)], v)
                    if do_cnt:
                        plsc.addupdate(
                            cnt_v.at[g >> 3, pl.ds((g & 7) * 16, 16)], ones16
                        )
                return carry2

            lax.fori_loop(0, CH // 16, group_body, 0)
            return carry

        lax.fori_loop(0, nj, do_chunk, 0)

    def zero_cnt(_i, carry):
        for kk in range(HALF // 16):
            cnt_v[_i, pl.ds(kk * 16, 16)] = zero16
        return carry

    lax.fori_loop(0, CNT_ROWS, zero_cnt, 0)
    half_pass(hlo_hbm, True)
    pltpu.sync_copy(acc_v, part_out.at[0, wid])
    pltpu.sync_copy(cnt_v, cnt_out.at[wid])
    half_pass(hhi_hbm, False)
    pltpu.sync_copy(acc_v, part_out.at[1, wid])


def _segment_sums(h_lo, h_hi, ids2d):
    mesh = plsc.VectorSubcoreMesh(core_axis_name="c", subcore_axis_name="s")
    f = pl.kernel(
        _seg_body,
        out_type=(
            jax.ShapeDtypeStruct((2, N_WORKERS, GPAD, HALF), jnp.float32),
            jax.ShapeDtypeStruct((N_WORKERS, CNT_ROWS, HALF), jnp.float32),
        ),
        mesh=mesh,
        scratch_types=[
            pltpu.VMEM((GPAD, HALF), jnp.float32),
            pltpu.VMEM((CH, HALF), jnp.float32),
            pltpu.VMEM((CH,), jnp.int32),
            pltpu.VMEM((CNT_ROWS, HALF), jnp.float32),
        ],
    )
    return f(h_lo, h_hi, ids2d)


def _final_body(part_ref, cnt_ref, w3a_ref, w3b_ref, b3_ref, o_ref):
    p = part_ref[...]
    z1 = jnp.concatenate(
        [jnp.sum(p[0], axis=0), jnp.sum(p[1], axis=0)], axis=-1
    )
    s = jnp.sum(cnt_ref[...], axis=0)
    cnt = s[:GPAD, 0:1]
    z2 = z1 / jnp.maximum(cnt, 1.0)
    o_ref[...] = (
        jnp.dot(z1, w3a_ref[...], preferred_element_type=jnp.float32)
        + jnp.dot(z2, w3b_ref[...], preferred_element_type=jnp.float32)
        + b3_ref[...]
    )


def _readout(part, cntp, W3, b3):
    out = pl.pallas_call(
        _final_body,
        out_shape=jax.ShapeDtypeStruct((GPAD, DIM), jnp.float32),
    )(part, cntp.reshape(N_WORKERS, CNT_ROWS * 8, 16), W3[:DIM], W3[DIM:],
      b3.reshape(1, DIM))
    return out[:N_GRAPHS]


def kernel(x, node2graph, W1, b1, W2, b2, W3, b3):
    h_lo, h_hi = _gated_h(x, W1, b1, W2, b2)
    ids = node2graph.astype(jnp.int32)
    ids_pad = jnp.concatenate(
        [ids, jnp.full((N_PAD - N_NODES,), N_GRAPHS, jnp.int32)]
    ).reshape(N_CHUNKS, CH)
    part, cntp = _segment_sums(h_lo, h_hi, ids_pad)
    return _readout(part, cntp, W3, b3)


# speedup vs baseline: 2.5040x; 1.1982x over previous
import jax
import jax.numpy as jnp
from jax import lax
from jax.experimental import pallas as pl
from jax.experimental.pallas import tpu as pltpu
from jax.experimental.pallas import tpu_sc as plsc

N_NODES = 50000
DIM = 256
HALF = 128
N_GRAPHS = 500

ROW_BLK = 512
CH = 128
N_GATE = 50176
N_PAD = 57344
N_CHUNKS = N_PAD // CH
N_WORKERS = 32
NJ = N_CHUNKS // N_WORKERS
GPAD = 504
CNT_ROWS = 64
NLANE = HALF // 16


def _gate_body(x_ref, w1_ref, b1_ref, w2_ref, b2_ref, lo_ref, hi_ref):
    x = x_ref[...]
    h1 = jnp.dot(x, w1_ref[...], preferred_element_type=jnp.float32) + b1_ref[...]
    h2 = jnp.dot(x, w2_ref[...], preferred_element_type=jnp.float32) + b2_ref[...]
    h = h1 * jax.nn.sigmoid(h2)
    lo_ref[...] = h[:, :HALF]
    hi_ref[...] = h[:, HALF:]


def _gated_h(x, W1, b1, W2, b2):
    grid = (N_GATE // ROW_BLK,)
    return pl.pallas_call(
        _gate_body,
        grid=grid,
        in_specs=[
            pl.BlockSpec((ROW_BLK, DIM), lambda i: (i, 0)),
            pl.BlockSpec((DIM, DIM), lambda i: (0, 0)),
            pl.BlockSpec((1, DIM), lambda i: (0, 0)),
            pl.BlockSpec((DIM, DIM), lambda i: (0, 0)),
            pl.BlockSpec((1, DIM), lambda i: (0, 0)),
        ],
        out_specs=[
            pl.BlockSpec((ROW_BLK, HALF), lambda i: (i, 0)),
            pl.BlockSpec((ROW_BLK, HALF), lambda i: (i, 0)),
        ],
        out_shape=[
            jax.ShapeDtypeStruct((N_PAD, HALF), jnp.float32),
            jax.ShapeDtypeStruct((N_PAD, HALF), jnp.float32),
        ],
    )(x, W1, b1.reshape(1, DIM), W2, b2.reshape(1, DIM))


def _seg_body(hlo_hbm, hhi_hbm, ids_hbm,
              part_out, cnt_out,
              acc_v, h0_v, ids_v, cnt_v):
    cid = lax.axis_index("c")
    sid = lax.axis_index("s")
    wid = sid * 2 + cid
    base = wid * NJ
    ones16 = jnp.ones((16,), jnp.float32)
    sixteen16 = jnp.full((16,), 16.0, jnp.float32)
    zero16 = jnp.zeros((16,), jnp.float32)

    def zero_acc(_i, c):
        for kk in range(NLANE):
            acc_v[_i, pl.ds(kk * 16, 16)] = zero16
        return c

    def zero_cnt(_i, c):
        for kk in range(NLANE):
            cnt_v[_i, pl.ds(kk * 16, 16)] = zero16
        return c

    def half_pass(src_hbm, do_cnt):
        lax.fori_loop(0, GPAD, zero_acc, 0)
        if do_cnt:
            lax.fori_loop(0, CNT_ROWS, zero_cnt, 0)

        def do_chunk(j, buf):
            pltpu.sync_copy(ids_hbm.at[base + j], ids_v)
            pltpu.sync_copy(src_hbm.at[pl.ds((base + j) * CH, CH)], buf)

            def group_body(rg, c, buf=buf):
                ids16 = ids_v[pl.ds(rg * 16, 16)]
                g0 = ids16[0]
                g15 = ids16[15]
                r0 = rg * 16

                @pl.when(g0 == g15)
                def _():
                    for kk in range(NLANE):
                        vs = [
                            buf[r0 + l, pl.ds(kk * 16, 16)] for l in range(16)
                        ]
                        while len(vs) > 1:
                            vs = [
                                vs[m] + vs[m + 1] for m in range(0, len(vs), 2)
                            ]
                        plsc.addupdate(acc_v.at[g0, pl.ds(kk * 16, 16)], vs[0])
                    if do_cnt:
                        plsc.addupdate(
                            cnt_v.at[g0 >> 3, pl.ds((g0 & 7) * 16, 16)],
                            sixteen16,
                        )

                @pl.when(g0 != g15)
                def _():
                    for l in range(16):
                        g = ids16[l]
                        for kk in range(NLANE):
                            v = buf[r0 + l, pl.ds(kk * 16, 16)]
                            plsc.addupdate(acc_v.at[g, pl.ds(kk * 16, 16)], v)
                        if do_cnt:
                            plsc.addupdate(
                                cnt_v.at[g >> 3, pl.ds((g & 7) * 16, 16)],
                                ones16,
                            )
                return c

            lax.fori_loop(0, CH // 16, group_body, 0)

        def chunk_body(j, c):
            do_chunk(j, h0_v)
            return c

        lax.fori_loop(0, NJ, chunk_body, 0)

    half_pass(hlo_hbm, True)
    pltpu.sync_copy(acc_v, part_out.at[0, wid])
    pltpu.sync_copy(cnt_v, cnt_out.at[wid])
    half_pass(hhi_hbm, False)
    pltpu.sync_copy(acc_v, part_out.at[1, wid])


def _segment_sums(h_lo, h_hi, ids2d):
    mesh = plsc.VectorSubcoreMesh(core_axis_name="c", subcore_axis_name="s")
    f = pl.kernel(
        _seg_body,
        out_type=(
            jax.ShapeDtypeStruct((2, N_WORKERS, GPAD, HALF), jnp.float32),
            jax.ShapeDtypeStruct((N_WORKERS, CNT_ROWS, HALF), jnp.float32),
        ),
        mesh=mesh,
        scratch_types=[
            pltpu.VMEM((GPAD, HALF), jnp.float32),
            pltpu.VMEM((CH, HALF), jnp.float32),
            pltpu.VMEM((CH,), jnp.int32),
            pltpu.VMEM((CNT_ROWS, HALF), jnp.float32),
        ],
    )
    return f(h_lo, h_hi, ids2d)


def _final_body(part_ref, cnt_ref, w3a_ref, w3b_ref, b3_ref, o_ref):
    p = part_ref[...]
    z1 = jnp.concatenate(
        [jnp.sum(p[0], axis=0), jnp.sum(p[1], axis=0)], axis=-1
    )
    s = jnp.sum(cnt_ref[...], axis=0)
    cnt = s[:GPAD, 0:1]
    z2 = z1 / jnp.maximum(cnt, 1.0)
    o_ref[...] = (
        jnp.dot(z1, w3a_ref[...], preferred_element_type=jnp.float32)
        + jnp.dot(z2, w3b_ref[...], preferred_element_type=jnp.float32)
        + b3_ref[...]
    )


def _readout(part, cntp, W3, b3):
    out = pl.pallas_call(
        _final_body,
        out_shape=jax.ShapeDtypeStruct((GPAD, DIM), jnp.float32),
    )(part, cntp.reshape(N_WORKERS, CNT_ROWS * 8, 16), W3[:DIM], W3[DIM:],
      b3.reshape(1, DIM))
    return out[:N_GRAPHS]


def kernel(x, node2graph, W1, b1, W2, b2, W3, b3):
    h_lo, h_hi = _gated_h(x, W1, b1, W2, b2)
    ids = node2graph.astype(jnp.int32)
    ids_pad = jnp.concatenate(
        [ids, jnp.full((N_PAD - N_NODES,), N_GRAPHS, jnp.int32)]
    ).reshape(N_CHUNKS, CH)
    part, cntp = _segment_sums(h_lo, h_hi, ids_pad)
    return _readout(part, cntp, W3, b3)


# speedup vs baseline: 2.9269x; 1.1689x over previous
import jax
import jax.numpy as jnp
from jax import lax
from jax.experimental import pallas as pl
from jax.experimental.pallas import tpu as pltpu
from jax.experimental.pallas import tpu_sc as plsc

N_NODES = 50000
DIM = 256
HALF = 128
N_GRAPHS = 500

ROW_BLK = 512
CH = 128
N_GATE = 50176
N_PAD = 57344
N_CHUNKS = N_PAD // CH
N_WORKERS = 32
NJ = N_CHUNKS // N_WORKERS
GPAD = 504
CNT_ROWS = 64
NLANE = HALF // 16


def _gate_body(x_ref, w1_ref, b1_ref, w2_ref, b2_ref, lo_ref, hi_ref):
    x = x_ref[...]
    h1 = jnp.dot(x, w1_ref[...], preferred_element_type=jnp.float32) + b1_ref[...]
    h2 = jnp.dot(x, w2_ref[...], preferred_element_type=jnp.float32) + b2_ref[...]
    h = h1 * jax.nn.sigmoid(h2)
    lo_ref[...] = h[:, :HALF]
    hi_ref[...] = h[:, HALF:]


def _gated_h(x, W1, b1, W2, b2):
    grid = (N_GATE // ROW_BLK,)
    return pl.pallas_call(
        _gate_body,
        grid=grid,
        in_specs=[
            pl.BlockSpec((ROW_BLK, DIM), lambda i: (i, 0)),
            pl.BlockSpec((DIM, DIM), lambda i: (0, 0)),
            pl.BlockSpec((1, DIM), lambda i: (0, 0)),
            pl.BlockSpec((DIM, DIM), lambda i: (0, 0)),
            pl.BlockSpec((1, DIM), lambda i: (0, 0)),
        ],
        out_specs=[
            pl.BlockSpec((ROW_BLK, HALF), lambda i: (i, 0)),
            pl.BlockSpec((ROW_BLK, HALF), lambda i: (i, 0)),
        ],
        out_shape=[
            jax.ShapeDtypeStruct((N_PAD, HALF), jnp.float32),
            jax.ShapeDtypeStruct((N_PAD, HALF), jnp.float32),
        ],
    )(x, W1, b1.reshape(1, DIM), W2, b2.reshape(1, DIM))


def _seg_body(hlo_hbm, hhi_hbm, ids_hbm,
              part_out, cnt_out,
              acc_v, h0_v, h1_v, ids_v, cnt_v, sem0, sem1):
    cid = lax.axis_index("c")
    sid = lax.axis_index("s")
    wid = sid * 2 + cid
    base = wid * NJ
    ones16 = jnp.ones((16,), jnp.float32)
    sixteen16 = jnp.full((16,), 16.0, jnp.float32)
    zero16 = jnp.zeros((16,), jnp.float32)

    def zero_acc(_i, c):
        for kk in range(NLANE):
            acc_v[_i, pl.ds(kk * 16, 16)] = zero16
        return c

    def zero_cnt(_i, c):
        for kk in range(NLANE):
            cnt_v[_i, pl.ds(kk * 16, 16)] = zero16
        return c

    def half_pass(src_hbm, do_cnt):
        pltpu.async_copy(src_hbm.at[pl.ds(base * CH, CH)], h0_v, sem0)
        pltpu.async_copy(src_hbm.at[pl.ds((base + 1) * CH, CH)], h1_v, sem1)
        lax.fori_loop(0, GPAD, zero_acc, 0)
        if do_cnt:
            lax.fori_loop(0, CNT_ROWS, zero_cnt, 0)

        def do_chunk(j, buf, sem):
            pltpu.sync_copy(ids_hbm.at[base + j], ids_v)
            pltpu.make_async_copy(
                src_hbm.at[pl.ds(base * CH, CH)], buf, sem
            ).wait()

            def group_body(rg, c, buf=buf):
                ids16 = ids_v[pl.ds(rg * 16, 16)]
                g0 = ids16[0]
                g15 = ids16[15]
                r0 = rg * 16

                @pl.when(g0 == g15)
                def _():
                    for kk in range(NLANE):
                        vs = [
                            buf[r0 + l, pl.ds(kk * 16, 16)] for l in range(16)
                        ]
                        while len(vs) > 1:
                            vs = [
                                vs[m] + vs[m + 1] for m in range(0, len(vs), 2)
                            ]
                        plsc.addupdate(acc_v.at[g0, pl.ds(kk * 16, 16)], vs[0])
                    if do_cnt:
                        plsc.addupdate(
                            cnt_v.at[g0 >> 3, pl.ds((g0 & 7) * 16, 16)],
                            sixteen16,
                        )

                @pl.when(g0 != g15)
                def _():
                    for l in range(16):
                        g = ids16[l]
                        for kk in range(NLANE):
                            v = buf[r0 + l, pl.ds(kk * 16, 16)]
                            plsc.addupdate(acc_v.at[g, pl.ds(kk * 16, 16)], v)
                        if do_cnt:
                            plsc.addupdate(
                                cnt_v.at[g >> 3, pl.ds((g & 7) * 16, 16)],
                                ones16,
                            )
                return c

            lax.fori_loop(0, CH // 16, group_body, 0)

            @pl.when(j + 2 < NJ)
            def _():
                pltpu.async_copy(
                    src_hbm.at[pl.ds((base + j + 2) * CH, CH)], buf, sem
                )

        def pair_body(i, c):
            do_chunk(i * 2, h0_v, sem0)
            do_chunk(i * 2 + 1, h1_v, sem1)
            return c

        lax.fori_loop(0, NJ // 2, pair_body, 0)

    half_pass(hlo_hbm, True)
    pltpu.sync_copy(acc_v, part_out.at[0, wid])
    pltpu.sync_copy(cnt_v, cnt_out.at[wid])
    half_pass(hhi_hbm, False)
    pltpu.sync_copy(acc_v, part_out.at[1, wid])


def _segment_sums(h_lo, h_hi, ids2d):
    mesh = plsc.VectorSubcoreMesh(core_axis_name="c", subcore_axis_name="s")
    f = pl.kernel(
        _seg_body,
        out_type=(
            jax.ShapeDtypeStruct((2, N_WORKERS, GPAD, HALF), jnp.float32),
            jax.ShapeDtypeStruct((N_WORKERS, CNT_ROWS, HALF), jnp.float32),
        ),
        mesh=mesh,
        scratch_types=[
            pltpu.VMEM((GPAD, HALF), jnp.float32),
            pltpu.VMEM((CH, HALF), jnp.float32),
            pltpu.VMEM((CH, HALF), jnp.float32),
            pltpu.VMEM((CH,), jnp.int32),
            pltpu.VMEM((CNT_ROWS, HALF), jnp.float32),
            pltpu.SemaphoreType.DMA,
            pltpu.SemaphoreType.DMA,
        ],
    )
    return f(h_lo, h_hi, ids2d)


def _final_body(part_ref, cnt_ref, w3a_ref, w3b_ref, b3_ref, o_ref):
    p = part_ref[...]
    z1 = jnp.concatenate(
        [jnp.sum(p[0], axis=0), jnp.sum(p[1], axis=0)], axis=-1
    )
    s = jnp.sum(cnt_ref[...], axis=0)
    cnt = s[:GPAD, 0:1]
    z2 = z1 / jnp.maximum(cnt, 1.0)
    o_ref[...] = (
        jnp.dot(z1, w3a_ref[...], preferred_element_type=jnp.float32)
        + jnp.dot(z2, w3b_ref[...], preferred_element_type=jnp.float32)
        + b3_ref[...]
    )


def _readout(part, cntp, W3, b3):
    out = pl.pallas_call(
        _final_body,
        out_shape=jax.ShapeDtypeStruct((GPAD, DIM), jnp.float32),
    )(part, cntp.reshape(N_WORKERS, CNT_ROWS * 8, 16), W3[:DIM], W3[DIM:],
      b3.reshape(1, DIM))
    return out[:N_GRAPHS]


def kernel(x, node2graph, W1, b1, W2, b2, W3, b3):
    h_lo, h_hi = _gated_h(x, W1, b1, W2, b2)
    ids = node2graph.astype(jnp.int32)
    ids_pad = jnp.concatenate(
        [ids, jnp.full((N_PAD - N_NODES,), N_GRAPHS, jnp.int32)]
    ).reshape(N_CHUNKS, CH)
    part, cntp = _segment_sums(h_lo, h_hi, ids_pad)
    return _readout(part, cntp, W3, b3)


# speedup vs baseline: 3.1224x; 1.0668x over previous
import jax
import jax.numpy as jnp
from jax import lax
from jax.experimental import pallas as pl
from jax.experimental.pallas import tpu as pltpu
from jax.experimental.pallas import tpu_sc as plsc

N_NODES = 50000
DIM = 256
HALF = 128
N_GRAPHS = 500

ROW_BLK = 512
CH = 128
GRP = 16
N_GATE = 50176
N_PAD = 57344
N_CHUNKS = N_PAD // CH
N_BLOCKS = N_PAD // GRP
N_WORKERS = 32
NJ = N_CHUNKS // N_WORKERS
BPC = CH // GRP
GPAD = 504
CNT_ROWS = 64
NLANE = HALF // 16


def _gate_body(x_ref, w1_ref, b1_ref, w2_ref, b2_ref,
               lo_ref, hi_ref, bslo_ref, bshi_ref):
    x = x_ref[...]
    h1 = jnp.dot(x, w1_ref[...], preferred_element_type=jnp.float32) + b1_ref[...]
    h2 = jnp.dot(x, w2_ref[...], preferred_element_type=jnp.float32) + b2_ref[...]
    h = h1 * jax.nn.sigmoid(h2)
    lo = h[:, :HALF]
    hi = h[:, HALF:]
    lo_ref[...] = lo
    hi_ref[...] = hi
    col = lax.broadcasted_iota(jnp.int32, (ROW_BLK // GRP, ROW_BLK), 1)
    row = lax.broadcasted_iota(jnp.int32, (ROW_BLK // GRP, ROW_BLK), 0)
    agg = (col // GRP == row).astype(jnp.float32)
    bslo_ref[...] = jnp.dot(agg, lo, preferred_element_type=jnp.float32)
    bshi_ref[...] = jnp.dot(agg, hi, preferred_element_type=jnp.float32)


def _gated_h(x, W1, b1, W2, b2):
    grid = (N_GATE // ROW_BLK,)
    return pl.pallas_call(
        _gate_body,
        grid=grid,
        in_specs=[
            pl.BlockSpec((ROW_BLK, DIM), lambda i: (i, 0)),
            pl.BlockSpec((DIM, DIM), lambda i: (0, 0)),
            pl.BlockSpec((1, DIM), lambda i: (0, 0)),
            pl.BlockSpec((DIM, DIM), lambda i: (0, 0)),
            pl.BlockSpec((1, DIM), lambda i: (0, 0)),
        ],
        out_specs=[
            pl.BlockSpec((ROW_BLK, HALF), lambda i: (i, 0)),
            pl.BlockSpec((ROW_BLK, HALF), lambda i: (i, 0)),
            pl.BlockSpec((ROW_BLK // GRP, HALF), lambda i: (i, 0)),
            pl.BlockSpec((ROW_BLK // GRP, HALF), lambda i: (i, 0)),
        ],
        out_shape=[
            jax.ShapeDtypeStruct((N_PAD, HALF), jnp.float32),
            jax.ShapeDtypeStruct((N_PAD, HALF), jnp.float32),
            jax.ShapeDtypeStruct((N_BLOCKS, HALF), jnp.float32),
            jax.ShapeDtypeStruct((N_BLOCKS, HALF), jnp.float32),
        ],
    )(x, W1, b1.reshape(1, DIM), W2, b2.reshape(1, DIM))


def _seg_body(hlo_hbm, hhi_hbm, bslo_hbm, bshi_hbm, ids_hbm,
              part_out, cnt_out,
              acc_v, h0_v, h1_v, bs_v, ids_v, cnt_v, sem0, sem1, sem2):
    cid = lax.axis_index("c")
    sid = lax.axis_index("s")
    wid = sid * 2 + cid
    base = wid * NJ
    ones16 = jnp.ones((16,), jnp.float32)
    sixteen16 = jnp.full((16,), 16.0, jnp.float32)
    zero16 = jnp.zeros((16,), jnp.float32)

    def zero_acc(_i, c):
        for kk in range(NLANE):
            acc_v[_i, pl.ds(kk * 16, 16)] = zero16
        return c

    def zero_cnt(_i, c):
        for kk in range(NLANE):
            cnt_v[_i, pl.ds(kk * 16, 16)] = zero16
        return c

    def half_pass(src_hbm, bs_hbm, do_cnt):
        pltpu.async_copy(bs_hbm.at[pl.ds(base * BPC, NJ * BPC)], bs_v, sem2)
        pltpu.async_copy(src_hbm.at[pl.ds(base * CH, CH)], h0_v, sem0)
        pltpu.async_copy(src_hbm.at[pl.ds((base + 1) * CH, CH)], h1_v, sem1)
        lax.fori_loop(0, GPAD, zero_acc, 0)
        if do_cnt:
            lax.fori_loop(0, CNT_ROWS, zero_cnt, 0)
        pltpu.make_async_copy(
            bs_hbm.at[pl.ds(base * BPC, NJ * BPC)], bs_v, sem2
        ).wait()

        def do_chunk(j, buf, sem):
            pltpu.sync_copy(ids_hbm.at[base + j], ids_v)
            pltpu.make_async_copy(
                src_hbm.at[pl.ds(base * CH, CH)], buf, sem
            ).wait()

            def group_body(rg, c, buf=buf):
                ids16 = ids_v[pl.ds(rg * 16, 16)]
                g0 = ids16[0]
                g15 = ids16[15]
                r0 = rg * 16

                @pl.when(g0 == g15)
                def _():
                    for kk in range(NLANE):
                        v = bs_v[j * BPC + rg, pl.ds(kk * 16, 16)]
                        plsc.addupdate(acc_v.at[g0, pl.ds(kk * 16, 16)], v)
                    if do_cnt:
                        plsc.addupdate(
                            cnt_v.at[g0 >> 3, pl.ds((g0 & 7) * 16, 16)],
                            sixteen16,
                        )

                @pl.when(g0 != g15)
                def _():
                    for l in range(16):
                        g = ids16[l]
                        for kk in range(NLANE):
                            v = buf[r0 + l, pl.ds(kk * 16, 16)]
                            plsc.addupdate(acc_v.at[g, pl.ds(kk * 16, 16)], v)
                        if do_cnt:
                            plsc.addupdate(
                                cnt_v.at[g >> 3, pl.ds((g & 7) * 16, 16)],
                                ones16,
                            )
                return c

            lax.fori_loop(0, CH // 16, group_body, 0)

            @pl.when(j + 2 < NJ)
            def _():
                pltpu.async_copy(
                    src_hbm.at[pl.ds((base + j + 2) * CH, CH)], buf, sem
                )

        def pair_body(i, c):
            do_chunk(i * 2, h0_v, sem0)
            do_chunk(i * 2 + 1, h1_v, sem1)
            return c

        lax.fori_loop(0, NJ // 2, pair_body, 0)

    half_pass(hlo_hbm, bslo_hbm, True)
    pltpu.sync_copy(acc_v, part_out.at[0, wid])
    pltpu.sync_copy(cnt_v, cnt_out.at[wid])
    half_pass(hhi_hbm, bshi_hbm, False)
    pltpu.sync_copy(acc_v, part_out.at[1, wid])


def _segment_sums(h_lo, h_hi, bs_lo, bs_hi, ids2d):
    mesh = plsc.VectorSubcoreMesh(core_axis_name="c", subcore_axis_name="s")
    f = pl.kernel(
        _seg_body,
        out_type=(
            jax.ShapeDtypeStruct((2, N_WORKERS, GPAD, HALF), jnp.float32),
            jax.ShapeDtypeStruct((N_WORKERS, CNT_ROWS, HALF), jnp.float32),
        ),
        mesh=mesh,
        scratch_types=[
            pltpu.VMEM((GPAD, HALF), jnp.float32),
            pltpu.VMEM((CH, HALF), jnp.float32),
            pltpu.VMEM((CH, HALF), jnp.float32),
            pltpu.VMEM((NJ * BPC, HALF), jnp.float32),
            pltpu.VMEM((CH,), jnp.int32),
            pltpu.VMEM((CNT_ROWS, HALF), jnp.float32),
            pltpu.SemaphoreType.DMA,
            pltpu.SemaphoreType.DMA,
            pltpu.SemaphoreType.DMA,
        ],
    )
    return f(h_lo, h_hi, bs_lo, bs_hi, ids2d)


def _final_body(part_ref, cnt_ref, w3a_ref, w3b_ref, b3_ref, o_ref):
    p = part_ref[...]
    z1 = jnp.concatenate(
        [jnp.sum(p[0], axis=0), jnp.sum(p[1], axis=0)], axis=-1
    )
    s = jnp.sum(cnt_ref[...], axis=0)
    cnt = s[:GPAD, 0:1]
    z2 = z1 / jnp.maximum(cnt, 1.0)
    o_ref[...] = (
        jnp.dot(z1, w3a_ref[...], preferred_element_type=jnp.float32)
        + jnp.dot(z2, w3b_ref[...], preferred_element_type=jnp.float32)
        + b3_ref[...]
    )


def _readout(part, cntp, W3, b3):
    out = pl.pallas_call(
        _final_body,
        out_shape=jax.ShapeDtypeStruct((GPAD, DIM), jnp.float32),
    )(part, cntp.reshape(N_WORKERS, CNT_ROWS * 8, 16), W3[:DIM], W3[DIM:],
      b3.reshape(1, DIM))
    return out[:N_GRAPHS]


def kernel(x, node2graph, W1, b1, W2, b2, W3, b3):
    h_lo, h_hi, bs_lo, bs_hi = _gated_h(x, W1, b1, W2, b2)
    ids = node2graph.astype(jnp.int32)
    ids_pad = jnp.concatenate(
        [ids, jnp.full((N_PAD - N_NODES,), N_GRAPHS, jnp.int32)]
    ).reshape(N_CHUNKS, CH)
    part, cntp = _segment_sums(h_lo, h_hi, bs_lo, bs_hi, ids_pad)
    return _readout(part, cntp, W3, b3)
